# Initial kernel scaffold; baseline (speedup 1.0000x reference)
import jax
import jax.numpy as jnp
from jax.experimental import pallas as pl


def kernel(x, edge_index, n, W, b, a_param):
    raise NotImplementedError("write your pallas kernel here")


# speedup vs baseline: 3.1052x; 3.1052x over previous
import functools

import numpy as np

import jax
import jax.numpy as jnp
from jax import lax
from jax.experimental import pallas as pl
from jax.experimental.pallas import tpu as pltpu
from jax.experimental.pallas import tpu_sc as plsc

N_PAD = 10240
NHALF = N_PAD // 2
NACC = 6400
CH = 64
CF = 128
ALPHA = 0.2
K = 80
NW = 32
ROWB = 256
NR16 = N_PAD // 16


def _proj_body(x_ref, wct_ref, bc_ref, p_ref, h0_ref, h1_ref, tab_ref):
    h = jnp.dot(x_ref[...], wct_ref[...], preferred_element_type=jnp.float32)
    h = h + bc_ref[...]
    s = jnp.dot(h, p_ref[...], preferred_element_type=jnp.float32)
    h0_ref[...] = h[:, 0:CH].astype(jnp.bfloat16)
    h1_ref[...] = h[:, CH:CF].astype(jnp.bfloat16)
    tab_ref[...] = s[:, 0:8]


def _final_body(p0_ref, p1_ref, rs0_ref, rs1_ref, out_ref):
    r0 = p0_ref[...] / rs0_ref[:, 0:1]
    r1 = p1_ref[...] / rs1_ref[:, 0:1]
    out_ref[:, 0:CH] = jnp.where(r0 > 0, r0, jnp.exp(r0) - 1.0)
    out_ref[:, CH:CF] = jnp.where(r1 > 0, r1, jnp.exp(r1) - 1.0)


def _make_edge_kernel(E, head):
    EPW = E // 16
    NCHUNK = EPW // K
    ZPT = NACC // 16
    WPT = NHALF // 16

    mesh = plsc.VectorSubcoreMesh(core_axis_name="c", subcore_axis_name="s")

    @functools.partial(
        pl.kernel,
        out_type=[
            jax.ShapeDtypeStruct((N_PAD, CH), jnp.float32),
            jax.ShapeDtypeStruct((N_PAD, 16), jnp.float32),
        ],
        mesh=mesh,
        scratch_types=[
            pltpu.VMEM((N_PAD, 8), jnp.float32),
            pltpu.VMEM((K,), jnp.int32),
            pltpu.VMEM((K,), jnp.int32),
            pltpu.VMEM((K,), jnp.int32),
            pltpu.VMEM((K,), jnp.float32),
            pltpu.VMEM((K, CH), jnp.bfloat16),
            pltpu.VMEM((K, CH), jnp.float32),
            pltpu.VMEM((NR16, 16), jnp.float32),
            pltpu.VMEM((5, 128), jnp.int32),
            pltpu.VMEM((K, 16), jnp.float32),
            pltpu.VMEM_SHARED((NACC, CH), jnp.float32),
            pltpu.VMEM_SHARED((NR16, 16), jnp.float32),
            pltpu.SemaphoreType.DMA,
        ],
        compiler_params=pltpu.CompilerParams(
            use_tc_tiling_on_sc=False, needs_layout_passes=False),
    )
    def edge_kernel(hext_hbm, tab_hbm, ei_hbm, part_hbm, rs_hbm,
                    tab, src_v, dst_v, sidx_v, wbuf, rows_bf, rows_v, trs,
                    ibuf, rsw_v, acc, rs_acc, sem):
        cid = lax.axis_index("c")
        sid = lax.axis_index("s")
        lane = lax.iota(jnp.int32, 16)
        zeros16 = jnp.zeros((16,), jnp.float32)
        ebase = sid * EPW
        lo = cid * NHALF

        pltpu.sync_copy(tab_hbm, tab)
        for j in range(5):
            for g in range(8):
                ibuf[j, pl.ds(g * 16, 16)] = j * 128 + g * 16 + lane

        def zrow(r, carry):
            for c in range(CH // 16):
                rows_v[r, pl.ds(c * 16, 16)] = zeros16
            return carry
        lax.fori_loop(0, K, zrow, 0)

        def zrs(r, carry):
            trs[r, pl.ds(0, 16)] = zeros16
            return carry
        lax.fori_loop(0, NR16, zrs, 0)

        def zrw(r, carry):
            rsw_v[r, pl.ds(0, 16)] = zeros16
            return carry
        lax.fori_loop(0, K, zrw, 0)

        for z in range(ZPT // K):
            pltpu.sync_copy(rows_v, acc.at[pl.ds(sid * ZPT + z * K, K)])
        pltpu.sync_copy(trs.at[pl.ds(0, NR16 // 16)],
                        rs_acc.at[pl.ds(sid * (NR16 // 16), NR16 // 16)])
        plsc.subcore_barrier()

        def chunk(k, carry):
            base = ebase + k * K
            pltpu.sync_copy(ei_hbm.at[pl.ds(base, K)], src_v)
            pltpu.sync_copy(ei_hbm.at[pl.ds(E + base, K)], dst_v)
            pltpu.async_copy(hext_hbm.at[dst_v], rows_bf, sem).wait()
            dcol = jnp.full((16,), head, jnp.int32)
            scol = jnp.full((16,), 2 + head, jnp.int32)
            for g in range(K // 16):
                src16 = src_v[pl.ds(g * 16, 16)]
                dst16 = dst_v[pl.ds(g * 16, 16)]
                sd = plsc.load_gather(tab, [dst16, dcol])
                ss = plsc.load_gather(tab, [src16, scol])
                l = ss + sd
                w = jnp.exp(jnp.where(l >= 0.0, -l, -ALPHA * l))
                t = src16 - lo
                own = (t >= 0) & (t < NHALF)
                w = jnp.where(own, w, 0.0)
                sidx_v[pl.ds(g * 16, 16)] = jnp.where(own, t, NHALF)
                wbuf[pl.ds(g * 16, 16)] = w
                plsc.addupdate_scatter(
                    trs, [lax.shift_right_logical(src16, 4),
                          jnp.bitwise_and(src16, 15)], w)

            def scale(e, carry2):
                ws = plsc.load_gather(wbuf, [jnp.full((16,), e, jnp.int32)])
                for c in range(CH // 32):
                    v = rows_bf[e, pl.ds(c * 32, 32)]
                    xi = plsc.bitcast(v, jnp.int32)
                    fe = plsc.bitcast(lax.shift_left(xi, 16), jnp.float32)
                    fo = plsc.bitcast(
                        jnp.bitwise_and(xi, jnp.int32(-65536)), jnp.float32)
                    rows_v[e, pl.ds(c * 32, 16)] = fe * ws
                    rows_v[e, pl.ds(c * 32 + 16, 16)] = fo * ws
                return carry2
            lax.fori_loop(0, K, scale, 0)

            pltpu.sync_copy(rows_v, acc.at[sidx_v], add=True)
            return carry
        lax.fori_loop(0, NCHUNK, chunk, 0)
        plsc.subcore_barrier()

        for j in range(5):
            pltpu.sync_copy(trs.at[pl.ds(j * 128, 128)],
                            rs_acc.at[ibuf.at[j]], add=True)
        plsc.subcore_barrier()

        pltpu.sync_copy(rs_acc.at[pl.ds(cid * (NHALF // 16) + sid * 20, 20)],
                        trs.at[pl.ds(0, 20)])
        for z in range(WPT // K):
            r0 = sid * WPT + z * K
            pltpu.sync_copy(acc.at[pl.ds(r0, K)], rows_v)
            pltpu.sync_copy(rows_v, part_hbm.at[pl.ds(lo + r0, K)])
            for q in range(K // 16):
                rsv = trs[z * (K // 16) + q, pl.ds(0, 16)]
                plsc.store_scatter(
                    rsw_v, [jnp.full((16,), q * 16, jnp.int32) + lane,
                            jnp.zeros((16,), jnp.int32)], rsv)
            pltpu.sync_copy(rsw_v, rs_hbm.at[pl.ds(lo + r0, K)])

    return edge_kernel


def kernel(x, edge_index, n, W, b, a_param):
    nodes = x.shape[0]
    E = edge_index.shape[1]
    nh, ch, cin = W.shape

    wct = W.reshape(nh * ch, cin).T
    bc = b.reshape(1, nh * ch)
    a0, a1 = a_param[0], a_param[1]
    z = jnp.zeros((CH,), jnp.float32)
    cols = [
        jnp.concatenate([a0[CH:], z]),
        jnp.concatenate([z, a1[CH:]]),
        jnp.concatenate([a0[:CH], z]),
        jnp.concatenate([z, a1[:CH]]),
    ] + [jnp.zeros((CF,), jnp.float32)] * 28
    P = jnp.stack(cols, axis=1)

    perm = np.array([(m // 32) * 32 + (m % 32) // 2 + (16 if m % 2 else 0)
                     for m in range(CF)], dtype=np.int32)
    wct = wct[:, perm]
    bc = bc[:, perm]
    P = P[perm, :]
    x_pad = jnp.pad(x, ((0, N_PAD - nodes), (0, 0)))

    grid = N_PAD // ROWB
    h0, h1, tab = pl.pallas_call(
        _proj_body,
        grid=(grid,),
        in_specs=[
            pl.BlockSpec((ROWB, CF), lambda i: (i, 0)),
            pl.BlockSpec((CF, CF), lambda i: (0, 0)),
            pl.BlockSpec((1, CF), lambda i: (0, 0)),
            pl.BlockSpec((CF, 32), lambda i: (0, 0)),
        ],
        out_specs=[
            pl.BlockSpec((ROWB, CH), lambda i: (i, 0)),
            pl.BlockSpec((ROWB, CH), lambda i: (i, 0)),
            pl.BlockSpec((ROWB, 8), lambda i: (i, 0)),
        ],
        out_shape=[
            jax.ShapeDtypeStruct((N_PAD, CH), jnp.bfloat16),
            jax.ShapeDtypeStruct((N_PAD, CH), jnp.bfloat16),
            jax.ShapeDtypeStruct((N_PAD, 8), jnp.float32),
        ],
    )(x_pad, wct, bc, P)

    ei_flat = edge_index.reshape(-1)
    part0, rs0 = _make_edge_kernel(E, 0)(h0, tab, ei_flat)
    part1, rs1 = _make_edge_kernel(E, 1)(h1, tab, ei_flat)

    out_pad = pl.pallas_call(
        _final_body,
        grid=(grid,),
        in_specs=[
            pl.BlockSpec((ROWB, CH), lambda i: (i, 0)),
            pl.BlockSpec((ROWB, CH), lambda i: (i, 0)),
            pl.BlockSpec((ROWB, 16), lambda i: (i, 0)),
            pl.BlockSpec((ROWB, 16), lambda i: (i, 0)),
        ],
        out_specs=pl.BlockSpec((ROWB, CF), lambda i: (i, 0)),
        out_shape=jax.ShapeDtypeStruct((N_PAD, CF), jnp.float32),
    )(part0, part1, rs0, rs1)

    return out_pad[:nodes]


# speedup vs baseline: 4.3303x; 1.3945x over previous
import functools

import numpy as np

import jax
import jax.numpy as jnp
from jax import lax
from jax.experimental import pallas as pl
from jax.experimental.pallas import tpu as pltpu
from jax.experimental.pallas import tpu_sc as plsc

N_PAD = 10240
NHALF = N_PAD // 2
NACC = 5136
CH = 64
CF = 128
ALPHA = 0.2
K = 80
NW = 32
ROWB = 256
NR16 = N_PAD // 16


def _proj_body(x_ref, wct_ref, bc_ref, p_ref, h0_ref, h1_ref, tab_ref):
    h = jnp.dot(x_ref[...], wct_ref[...], preferred_element_type=jnp.float32)
    h = h + bc_ref[...]
    s = jnp.dot(h, p_ref[...], preferred_element_type=jnp.float32)
    h0_ref[...] = h[:, 0:CH].astype(jnp.bfloat16)
    h1_ref[...] = h[:, CH:CF].astype(jnp.bfloat16)
    tab_ref[...] = s[:, 0:8]


def _final_body(p0_ref, p1_ref, rs0_ref, rs1_ref, out_ref):
    r0 = p0_ref[...] / rs0_ref[:, 0:1]
    r1 = p1_ref[...] / rs1_ref[:, 0:1]
    out_ref[:, 0:CH] = jnp.where(r0 > 0, r0, jnp.exp(r0) - 1.0)
    out_ref[:, CH:CF] = jnp.where(r1 > 0, r1, jnp.exp(r1) - 1.0)


def _make_edge_kernel(E, head):
    EPW = E // 16
    NCHUNK = EPW // K
    ZPT = NACC // 16
    WPT = NHALF // 16

    mesh = plsc.VectorSubcoreMesh(core_axis_name="c", subcore_axis_name="s")

    @functools.partial(
        pl.kernel,
        out_type=[
            jax.ShapeDtypeStruct((N_PAD, CH), jnp.float32),
            jax.ShapeDtypeStruct((N_PAD, 16), jnp.float32),
        ],
        mesh=mesh,
        scratch_types=[
            pltpu.VMEM((N_PAD, 8), jnp.float32),
            [pltpu.VMEM((K,), jnp.int32)] * 2,
            [pltpu.VMEM((K,), jnp.int32)] * 2,
            [pltpu.VMEM((K,), jnp.int32)] * 2,
            [pltpu.VMEM((K,), jnp.float32)] * 2,
            [pltpu.VMEM((K, CH), jnp.bfloat16)] * 2,
            [pltpu.VMEM((K, CH), jnp.float32)] * 2,
            pltpu.VMEM((NR16, 16), jnp.float32),
            pltpu.VMEM((4, 80), jnp.int32),
            pltpu.VMEM((K, 16), jnp.float32),
            pltpu.VMEM_SHARED((NACC, CH), jnp.float32),
            pltpu.VMEM_SHARED((NR16 // 2, 16), jnp.float32),
            [pltpu.SemaphoreType.DMA] * 2,
            [pltpu.SemaphoreType.DMA] * 2,
        ],
        compiler_params=pltpu.CompilerParams(
            use_tc_tiling_on_sc=False, needs_layout_passes=False),
    )
    def edge_kernel(hext_hbm, tab_hbm, ei_hbm, part_hbm, rs_hbm,
                    tab, src_b, dst_b, sidx_b, wbuf_b, rows_bf_b, rows_v_b,
                    trs, ibuf, rsw_v, acc, rs_acc, sem_g, sem_s):
        cid = lax.axis_index("c")
        sid = lax.axis_index("s")
        lane = lax.iota(jnp.int32, 16)
        zeros16 = jnp.zeros((16,), jnp.float32)
        ebase = sid * EPW
        lo = cid * NHALF

        pltpu.sync_copy(tab_hbm, tab)
        for j in range(4):
            for g in range(5):
                ibuf[j, pl.ds(g * 16, 16)] = j * 80 + g * 16 + lane

        def zrow(r, carry):
            for c in range(CH // 16):
                rows_v_b[0][r, pl.ds(c * 16, 16)] = zeros16
            return carry
        lax.fori_loop(0, K, zrow, 0)

        def zrs(r, carry):
            trs[r, pl.ds(0, 16)] = zeros16
            return carry
        lax.fori_loop(0, NR16, zrs, 0)

        def zrw(r, carry):
            rsw_v[r, pl.ds(0, 16)] = zeros16
            return carry
        lax.fori_loop(0, K, zrw, 0)

        for z in range(ZPT // K):
            pltpu.sync_copy(rows_v_b[0], acc.at[pl.ds(sid * ZPT + z * K, K)])
        pltpu.sync_copy(rows_v_b[0].at[pl.ds(0, ZPT % K)],
                        acc.at[pl.ds(sid * ZPT + (ZPT // K) * K, ZPT % K)])
        pltpu.sync_copy(trs.at[pl.ds(0, 20)],
                        rs_acc.at[pl.ds(sid * 20, 20)])
        plsc.subcore_barrier()

        dcol = jnp.full((16,), head, jnp.int32)
        scol = jnp.full((16,), 2 + head, jnp.int32)

        def load_idx(c, b):
            base = ebase + c * K
            pltpu.sync_copy(ei_hbm.at[pl.ds(base, K)], src_b[b])
            pltpu.sync_copy(ei_hbm.at[pl.ds(E + base, K)], dst_b[b])
            pltpu.async_copy(hext_hbm.at[dst_b[b]], rows_bf_b[b], sem_g[b])

        def compute(b):
            src_v, dst_v = src_b[b], dst_b[b]
            sidx_v, wbuf = sidx_b[b], wbuf_b[b]
            rows_bf, rows_v = rows_bf_b[b], rows_v_b[b]
            for g in range(K // 16):
                src16 = src_v[pl.ds(g * 16, 16)]
                dst16 = dst_v[pl.ds(g * 16, 16)]
                sd = plsc.load_gather(tab, [dst16, dcol])
                ss = plsc.load_gather(tab, [src16, scol])
                l = ss + sd
                w = jnp.exp(jnp.where(l >= 0.0, -l, -ALPHA * l))
                t = src16 - lo
                own = (t >= 0) & (t < NHALF)
                w = jnp.where(own, w, 0.0)
                sidx_v[pl.ds(g * 16, 16)] = jnp.where(own, t, NHALF)
                wbuf[pl.ds(g * 16, 16)] = w
                plsc.addupdate_scatter(
                    trs, [lax.shift_right_logical(src16, 4),
                          jnp.bitwise_and(src16, 15)], w)

            def scale(e, carry2):
                ws = plsc.load_gather(wbuf, [jnp.full((16,), e, jnp.int32)])
                for c in range(CH // 32):
                    v = rows_bf[e, pl.ds(c * 32, 32)]
                    xi = plsc.bitcast(v, jnp.int32)
                    fe = plsc.bitcast(lax.shift_left(xi, 16), jnp.float32)
                    fo = plsc.bitcast(
                        jnp.bitwise_and(xi, jnp.int32(-65536)), jnp.float32)
                    rows_v[e, pl.ds(c * 32, 16)] = fe * ws
                    rows_v[e, pl.ds(c * 32 + 16, 16)] = fo * ws
                return carry2
            lax.fori_loop(0, K, scale, 0)

        def wait_gather(b):
            pltpu.make_async_copy(
                hext_hbm.at[dst_b[b]], rows_bf_b[b], sem_g[b]).wait()

        def wait_scatter(b):
            pltpu.make_async_copy(
                rows_v_b[b], acc.at[sidx_b[b]], sem_s[b]).wait()

        def issue_scatter(b):
            pltpu.async_copy(rows_v_b[b], acc.at[sidx_b[b]], sem_s[b],
                             add=True)

        load_idx(0, 0)

        def pipe(kk, carry):
            c0 = 2 * kk
            wait_gather(0)
            load_idx(c0 + 1, 1)
            pl.when(kk > 0)(lambda: wait_scatter(0))
            compute(0)
            issue_scatter(0)
            wait_gather(1)
            pl.when(kk < NCHUNK // 2 - 1)(lambda: load_idx(c0 + 2, 0))
            pl.when(kk > 0)(lambda: wait_scatter(1))
            compute(1)
            issue_scatter(1)
            return carry
        lax.fori_loop(0, NCHUNK // 2, pipe, 0)
        wait_scatter(0)
        wait_scatter(1)
        plsc.subcore_barrier()

        for j in range(4):
            pltpu.sync_copy(trs.at[pl.ds(cid * (NHALF // 16) + j * 80, 80)],
                            rs_acc.at[ibuf.at[j]], add=True)
        plsc.subcore_barrier()

        pltpu.sync_copy(rs_acc.at[pl.ds(sid * 20, 20)],
                        trs.at[pl.ds(0, 20)])
        for z in range(WPT // K):
            r0 = sid * WPT + z * K
            pltpu.sync_copy(acc.at[pl.ds(r0, K)], rows_v_b[0])
            pltpu.sync_copy(rows_v_b[0], part_hbm.at[pl.ds(lo + r0, K)])
            for q in range(K // 16):
                rsv = trs[z * (K // 16) + q, pl.ds(0, 16)]
                plsc.store_scatter(
                    rsw_v, [jnp.full((16,), q * 16, jnp.int32) + lane,
                            jnp.zeros((16,), jnp.int32)], rsv)
            pltpu.sync_copy(rsw_v, rs_hbm.at[pl.ds(lo + r0, K)])

    return edge_kernel


def kernel(x, edge_index, n, W, b, a_param):
    nodes = x.shape[0]
    E = edge_index.shape[1]
    nh, ch, cin = W.shape

    wct = W.reshape(nh * ch, cin).T
    bc = b.reshape(1, nh * ch)
    a0, a1 = a_param[0], a_param[1]
    z = jnp.zeros((CH,), jnp.float32)
    cols = [
        jnp.concatenate([a0[CH:], z]),
        jnp.concatenate([z, a1[CH:]]),
        jnp.concatenate([a0[:CH], z]),
        jnp.concatenate([z, a1[:CH]]),
    ] + [jnp.zeros((CF,), jnp.float32)] * 28
    P = jnp.stack(cols, axis=1)

    perm = np.array([(m // 32) * 32 + (m % 32) // 2 + (16 if m % 2 else 0)
                     for m in range(CF)], dtype=np.int32)
    wct = wct[:, perm]
    bc = bc[:, perm]
    P = P[perm, :]
    x_pad = jnp.pad(x, ((0, N_PAD - nodes), (0, 0)))

    grid = N_PAD // ROWB
    h0, h1, tab = pl.pallas_call(
        _proj_body,
        grid=(grid,),
        in_specs=[
            pl.BlockSpec((ROWB, CF), lambda i: (i, 0)),
            pl.BlockSpec((CF, CF), lambda i: (0, 0)),
            pl.BlockSpec((1, CF), lambda i: (0, 0)),
            pl.BlockSpec((CF, 32), lambda i: (0, 0)),
        ],
        out_specs=[
            pl.BlockSpec((ROWB, CH), lambda i: (i, 0)),
            pl.BlockSpec((ROWB, CH), lambda i: (i, 0)),
            pl.BlockSpec((ROWB, 8), lambda i: (i, 0)),
        ],
        out_shape=[
            jax.ShapeDtypeStruct((N_PAD, CH), jnp.bfloat16),
            jax.ShapeDtypeStruct((N_PAD, CH), jnp.bfloat16),
            jax.ShapeDtypeStruct((N_PAD, 8), jnp.float32),
        ],
    )(x_pad, wct, bc, P)

    ei_flat = edge_index.reshape(-1)
    part0, rs0 = _make_edge_kernel(E, 0)(h0, tab, ei_flat)
    part1, rs1 = _make_edge_kernel(E, 1)(h1, tab, ei_flat)

    out_pad = pl.pallas_call(
        _final_body,
        grid=(grid,),
        in_specs=[
            pl.BlockSpec((ROWB, CH), lambda i: (i, 0)),
            pl.BlockSpec((ROWB, CH), lambda i: (i, 0)),
            pl.BlockSpec((ROWB, 16), lambda i: (i, 0)),
            pl.BlockSpec((ROWB, 16), lambda i: (i, 0)),
        ],
        out_specs=pl.BlockSpec((ROWB, CF), lambda i: (i, 0)),
        out_shape=jax.ShapeDtypeStruct((N_PAD, CF), jnp.float32),
    )(part0, part1, rs0, rs1)

    return out_pad[:nodes]


# speedup vs baseline: 4.8257x; 1.1144x over previous
import functools

import numpy as np

import jax
import jax.numpy as jnp
from jax import lax
from jax.experimental import pallas as pl
from jax.experimental.pallas import tpu as pltpu
from jax.experimental.pallas import tpu_sc as plsc

N_PAD = 10240
NHALF = N_PAD // 2
NACC = 5136
CH = 64
CF = 128
ALPHA = 0.2
K = 80
NW = 32
ROWB = 256
NR16 = N_PAD // 16


def _proj_body(x_ref, wct_ref, bc_ref, p_ref, h0_ref, h1_ref, tab_ref):
    h = jnp.dot(x_ref[...], wct_ref[...], preferred_element_type=jnp.float32)
    h = h + bc_ref[...]
    s = jnp.dot(h, p_ref[...], preferred_element_type=jnp.float32)
    h0_ref[...] = h[:, 0:CH].astype(jnp.bfloat16)
    h1_ref[...] = h[:, CH:CF].astype(jnp.bfloat16)
    tab_ref[...] = s[:, 0:8]


def _final_body(p0_ref, p1_ref, rs0_ref, rs1_ref, out_ref):
    r0 = p0_ref[...] / rs0_ref[:, 0:1]
    r1 = p1_ref[...] / rs1_ref[:, 0:1]
    out_ref[:, 0:CH] = jnp.where(r0 > 0, r0, jnp.exp(r0) - 1.0)
    out_ref[:, CH:CF] = jnp.where(r1 > 0, r1, jnp.exp(r1) - 1.0)


def _make_edge_kernel(E, head):
    EPW = E // 16
    NCHUNK = EPW // K
    ZPT = NACC // 16
    WPT = NHALF // 16

    mesh = plsc.VectorSubcoreMesh(core_axis_name="c", subcore_axis_name="s")

    @functools.partial(
        pl.kernel,
        out_type=[
            jax.ShapeDtypeStruct((N_PAD, CH), jnp.float32),
            jax.ShapeDtypeStruct((N_PAD, 16), jnp.float32),
        ],
        mesh=mesh,
        scratch_types=[
            pltpu.VMEM((N_PAD, 8), jnp.float32),
            [pltpu.VMEM((K,), jnp.int32)] * 2,
            [pltpu.VMEM((K,), jnp.int32)] * 2,
            [pltpu.VMEM((K,), jnp.int32)] * 2,
            [pltpu.VMEM((K,), jnp.float32)] * 2,
            [pltpu.VMEM((K, CH), jnp.bfloat16)] * 2,
            [pltpu.VMEM((K, CH), jnp.float32)] * 2,
            pltpu.VMEM((NR16, 16), jnp.float32),
            pltpu.VMEM((4, 80), jnp.int32),
            pltpu.VMEM((K, 16), jnp.float32),
            pltpu.VMEM_SHARED((NACC, CH), jnp.float32),
            pltpu.VMEM_SHARED((NR16 // 2, 16), jnp.float32),
            [pltpu.SemaphoreType.DMA] * 2,
            [pltpu.SemaphoreType.DMA] * 2,
            [pltpu.SemaphoreType.DMA] * 2,
        ],
        compiler_params=pltpu.CompilerParams(
            use_tc_tiling_on_sc=False, needs_layout_passes=False),
    )
    def edge_kernel(hext_hbm, tab_hbm, ei_hbm, part_hbm, rs_hbm,
                    tab, src_b, dst_b, sidx_b, wbuf_b, rows_bf_b, rows_v_b,
                    trs, ibuf, rsw_v, acc, rs_acc, sem_g, sem_s, sem_i):
        cid = lax.axis_index("c")
        sid = lax.axis_index("s")
        lane = lax.iota(jnp.int32, 16)
        zeros16 = jnp.zeros((16,), jnp.float32)
        ebase = sid * EPW
        lo = cid * NHALF

        pltpu.sync_copy(tab_hbm, tab)
        for j in range(4):
            for g in range(5):
                ibuf[j, pl.ds(g * 16, 16)] = j * 80 + g * 16 + lane

        def zrow(r, carry):
            for c in range(CH // 16):
                rows_v_b[0][r, pl.ds(c * 16, 16)] = zeros16
            return carry
        lax.fori_loop(0, K, zrow, 0)

        def zrs(r, carry):
            trs[r, pl.ds(0, 16)] = zeros16
            return carry
        lax.fori_loop(0, NR16, zrs, 0)

        def zrw(r, carry):
            rsw_v[r, pl.ds(0, 16)] = zeros16
            return carry
        lax.fori_loop(0, K, zrw, 0)

        for z in range(ZPT // K):
            pltpu.sync_copy(rows_v_b[0], acc.at[pl.ds(sid * ZPT + z * K, K)])
        pltpu.sync_copy(rows_v_b[0].at[pl.ds(0, ZPT % K)],
                        acc.at[pl.ds(sid * ZPT + (ZPT // K) * K, ZPT % K)])
        pltpu.sync_copy(trs.at[pl.ds(0, 20)],
                        rs_acc.at[pl.ds(sid * 20, 20)])
        plsc.subcore_barrier()

        dcol = jnp.full((16,), head, jnp.int32)
        scol = jnp.full((16,), 2 + head, jnp.int32)

        def issue_idx(c, b):
            base = ebase + c * K
            pltpu.async_copy(ei_hbm.at[pl.ds(base, K)], src_b[b], sem_i[b])
            pltpu.async_copy(ei_hbm.at[pl.ds(E + base, K)], dst_b[b], sem_i[b])

        def wait_idx(c, b):
            base = ebase + c * K
            pltpu.make_async_copy(
                ei_hbm.at[pl.ds(base, K)], src_b[b], sem_i[b]).wait()
            pltpu.make_async_copy(
                ei_hbm.at[pl.ds(E + base, K)], dst_b[b], sem_i[b]).wait()

        def issue_gather(b):
            pltpu.async_copy(hext_hbm.at[dst_b[b]], rows_bf_b[b], sem_g[b])

        def compute(b):
            src_v, dst_v = src_b[b], dst_b[b]
            sidx_v, wbuf = sidx_b[b], wbuf_b[b]
            rows_bf, rows_v = rows_bf_b[b], rows_v_b[b]
            for g in range(K // 16):
                src16 = src_v[pl.ds(g * 16, 16)]
                dst16 = dst_v[pl.ds(g * 16, 16)]
                sd = plsc.load_gather(tab, [dst16, dcol])
                ss = plsc.load_gather(tab, [src16, scol])
                l = ss + sd
                w = jnp.exp(jnp.where(l >= 0.0, -l, -ALPHA * l))
                t = src16 - lo
                own = (t >= 0) & (t < NHALF)
                w = jnp.where(own, w, 0.0)
                sidx_v[pl.ds(g * 16, 16)] = jnp.where(own, t, NHALF)
                wbuf[pl.ds(g * 16, 16)] = w
                plsc.addupdate_scatter(
                    trs, [lax.shift_right_logical(src16, 4),
                          jnp.bitwise_and(src16, 15)], w)

            def scale(e, carry2):
                ws = plsc.load_gather(wbuf, [jnp.full((16,), e, jnp.int32)])
                for c in range(CH // 32):
                    v = rows_bf[e, pl.ds(c * 32, 32)]
                    xi = plsc.bitcast(v, jnp.int32)
                    fe = plsc.bitcast(lax.shift_left(xi, 16), jnp.float32)
                    fo = plsc.bitcast(
                        jnp.bitwise_and(xi, jnp.int32(-65536)), jnp.float32)
                    rows_v[e, pl.ds(c * 32, 16)] = fe * ws
                    rows_v[e, pl.ds(c * 32 + 16, 16)] = fo * ws
                return carry2
            lax.fori_loop(0, K, scale, 0, unroll=4)

        def wait_gather(b):
            pltpu.make_async_copy(
                hext_hbm.at[dst_b[b]], rows_bf_b[b], sem_g[b]).wait()

        def wait_scatter(b):
            pltpu.make_async_copy(
                rows_v_b[b], acc.at[sidx_b[b]], sem_s[b]).wait()

        def issue_scatter(b):
            pltpu.async_copy(rows_v_b[b], acc.at[sidx_b[b]], sem_s[b],
                             add=True)

        issue_idx(0, 0)
        wait_idx(0, 0)
        issue_gather(0)
        issue_idx(1, 1)

        def pipe(kk, carry):
            c0 = 2 * kk
            wait_gather(0)
            pl.when(kk > 0)(lambda: wait_scatter(0))
            compute(0)
            issue_scatter(0)
            wait_idx(c0 + 1, 1)
            issue_gather(1)
            pl.when(kk < NCHUNK // 2 - 1)(lambda: issue_idx(c0 + 2, 0))
            wait_gather(1)
            pl.when(kk > 0)(lambda: wait_scatter(1))
            compute(1)
            issue_scatter(1)

            def _next0():
                wait_idx(c0 + 2, 0)
                issue_gather(0)
                pl.when(kk < NCHUNK // 2 - 1)(lambda: issue_idx(c0 + 3, 1))
            pl.when(kk < NCHUNK // 2 - 1)(_next0)
            return carry
        lax.fori_loop(0, NCHUNK // 2, pipe, 0)
        wait_scatter(0)
        wait_scatter(1)
        plsc.subcore_barrier()

        for j in range(4):
            pltpu.sync_copy(trs.at[pl.ds(cid * (NHALF // 16) + j * 80, 80)],
                            rs_acc.at[ibuf.at[j]], add=True)
        plsc.subcore_barrier()

        pltpu.sync_copy(rs_acc.at[pl.ds(sid * 20, 20)],
                        trs.at[pl.ds(0, 20)])
        for z in range(WPT // K):
            r0 = sid * WPT + z * K
            pltpu.sync_copy(acc.at[pl.ds(r0, K)], rows_v_b[0])
            pltpu.sync_copy(rows_v_b[0], part_hbm.at[pl.ds(lo + r0, K)])
            for q in range(K // 16):
                rsv = trs[z * (K // 16) + q, pl.ds(0, 16)]
                plsc.store_scatter(
                    rsw_v, [jnp.full((16,), q * 16, jnp.int32) + lane,
                            jnp.zeros((16,), jnp.int32)], rsv)
            pltpu.sync_copy(rsw_v, rs_hbm.at[pl.ds(lo + r0, K)])

    return edge_kernel


def kernel(x, edge_index, n, W, b, a_param):
    nodes = x.shape[0]
    E = edge_index.shape[1]
    nh, ch, cin = W.shape

    wct = W.reshape(nh * ch, cin).T
    bc = b.reshape(1, nh * ch)
    a0, a1 = a_param[0], a_param[1]
    z = jnp.zeros((CH,), jnp.float32)
    cols = [
        jnp.concatenate([a0[CH:], z]),
        jnp.concatenate([z, a1[CH:]]),
        jnp.concatenate([a0[:CH], z]),
        jnp.concatenate([z, a1[:CH]]),
    ] + [jnp.zeros((CF,), jnp.float32)] * 28
    P = jnp.stack(cols, axis=1)

    perm = np.array([(m // 32) * 32 + (m % 32) // 2 + (16 if m % 2 else 0)
                     for m in range(CF)], dtype=np.int32)
    wct = wct[:, perm]
    bc = bc[:, perm]
    P = P[perm, :]
    x_pad = jnp.pad(x, ((0, N_PAD - nodes), (0, 0)))

    grid = N_PAD // ROWB
    h0, h1, tab = pl.pallas_call(
        _proj_body,
        grid=(grid,),
        in_specs=[
            pl.BlockSpec((ROWB, CF), lambda i: (i, 0)),
            pl.BlockSpec((CF, CF), lambda i: (0, 0)),
            pl.BlockSpec((1, CF), lambda i: (0, 0)),
            pl.BlockSpec((CF, 32), lambda i: (0, 0)),
        ],
        out_specs=[
            pl.BlockSpec((ROWB, CH), lambda i: (i, 0)),
            pl.BlockSpec((ROWB, CH), lambda i: (i, 0)),
            pl.BlockSpec((ROWB, 8), lambda i: (i, 0)),
        ],
        out_shape=[
            jax.ShapeDtypeStruct((N_PAD, CH), jnp.bfloat16),
            jax.ShapeDtypeStruct((N_PAD, CH), jnp.bfloat16),
            jax.ShapeDtypeStruct((N_PAD, 8), jnp.float32),
        ],
    )(x_pad, wct, bc, P)

    ei_flat = edge_index.reshape(-1)
    part0, rs0 = _make_edge_kernel(E, 0)(h0, tab, ei_flat)
    part1, rs1 = _make_edge_kernel(E, 1)(h1, tab, ei_flat)

    out_pad = pl.pallas_call(
        _final_body,
        grid=(grid,),
        in_specs=[
            pl.BlockSpec((ROWB, CH), lambda i: (i, 0)),
            pl.BlockSpec((ROWB, CH), lambda i: (i, 0)),
            pl.BlockSpec((ROWB, 16), lambda i: (i, 0)),
            pl.BlockSpec((ROWB, 16), lambda i: (i, 0)),
        ],
        out_specs=pl.BlockSpec((ROWB, CF), lambda i: (i, 0)),
        out_shape=jax.ShapeDtypeStruct((N_PAD, CF), jnp.float32),
    )(part0, part1, rs0, rs1)

    return out_pad[:nodes]


# speedup vs baseline: 7.6017x; 1.5753x over previous
import functools

import numpy as np

import jax
import jax.numpy as jnp
from jax import lax
from jax.experimental import pallas as pl
from jax.experimental.pallas import tpu as pltpu
from jax.experimental.pallas import tpu_sc as plsc

N_PAD = 10240
NHALF = N_PAD // 2
NACC = 5136
CH = 64
CF = 128
ALPHA = 0.2
K = 80
NW = 32
ROWB = 256
NR16 = N_PAD // 16


def _proj_body(x_ref, wct_ref, bc_ref, p_ref, h0_ref, h1_ref, tab_ref):
    h = jnp.dot(x_ref[...], wct_ref[...], preferred_element_type=jnp.float32)
    h = h + bc_ref[...]
    s = jnp.dot(h, p_ref[...], preferred_element_type=jnp.float32)
    h0_ref[...] = h[:, 0:CH].astype(jnp.bfloat16)
    h1_ref[...] = h[:, CH:CF].astype(jnp.bfloat16)
    tab_ref[...] = s[:, 0:8]


def _final_body(p0_ref, p1_ref, rs0_ref, rs1_ref, out_ref):
    r0 = p0_ref[...] / rs0_ref[:, 0:1]
    r1 = p1_ref[...] / rs1_ref[:, 0:1]
    out_ref[:, 0:CH] = jnp.where(r0 > 0, r0, jnp.exp(r0) - 1.0)
    out_ref[:, CH:CF] = jnp.where(r1 > 0, r1, jnp.exp(r1) - 1.0)


def _make_edge_kernel(E, head):
    EPW = E // 16
    NCHUNK = EPW // K
    ZPT = NACC // 16
    WPT = NHALF // 16

    mesh = plsc.VectorSubcoreMesh(core_axis_name="c", subcore_axis_name="s")

    @functools.partial(
        pl.kernel,
        out_type=[
            jax.ShapeDtypeStruct((N_PAD, CH), jnp.float32),
            jax.ShapeDtypeStruct((N_PAD, 16), jnp.float32),
        ],
        mesh=mesh,
        scratch_types=[
            pltpu.VMEM((N_PAD, 8), jnp.float32),
            [pltpu.VMEM((K,), jnp.int32)] * 2,
            [pltpu.VMEM((K,), jnp.int32)] * 2,
            [pltpu.VMEM((K,), jnp.int32)] * 2,
            [pltpu.VMEM((K,), jnp.float32)] * 2,
            [pltpu.VMEM((K, CH), jnp.bfloat16)] * 2,
            [pltpu.VMEM((K, CH), jnp.float32)] * 2,
            pltpu.VMEM((NR16, 16), jnp.float32),
            pltpu.VMEM((4, 80), jnp.int32),
            pltpu.VMEM((K, 16), jnp.float32),
            pltpu.VMEM_SHARED((NACC, CH), jnp.float32),
            pltpu.VMEM_SHARED((NR16 // 2, 16), jnp.float32),
            [pltpu.SemaphoreType.DMA] * 2,
            [pltpu.SemaphoreType.DMA] * 2,
            [pltpu.SemaphoreType.DMA] * 2,
        ],
        compiler_params=pltpu.CompilerParams(
            use_tc_tiling_on_sc=False, needs_layout_passes=False),
    )
    def edge_kernel(hext_hbm, tab_hbm, ei_hbm, part_hbm, rs_hbm,
                    tab, src_b, dst_b, sidx_b, wbuf_b, rows_bf_b, rows_v_b,
                    trs, ibuf, rsw_v, acc, rs_acc, sem_g, sem_s, sem_i):
        cid = lax.axis_index("c")
        sid = lax.axis_index("s")
        lane = lax.iota(jnp.int32, 16)
        zeros16 = jnp.zeros((16,), jnp.float32)
        ebase = sid * EPW
        lo = cid * NHALF

        pltpu.sync_copy(tab_hbm, tab)
        for j in range(4):
            for g in range(5):
                ibuf[j, pl.ds(g * 16, 16)] = j * 80 + g * 16 + lane

        def zrow(r, carry):
            for c in range(CH // 16):
                rows_v_b[0][r, pl.ds(c * 16, 16)] = zeros16
            return carry
        lax.fori_loop(0, K, zrow, 0)

        def zrs(r, carry):
            trs[r, pl.ds(0, 16)] = zeros16
            return carry
        lax.fori_loop(0, NR16, zrs, 0)

        def zrw(r, carry):
            rsw_v[r, pl.ds(0, 16)] = zeros16
            return carry
        lax.fori_loop(0, K, zrw, 0)

        for z in range(ZPT // K):
            pltpu.sync_copy(rows_v_b[0], acc.at[pl.ds(sid * ZPT + z * K, K)])
        pltpu.sync_copy(rows_v_b[0].at[pl.ds(0, ZPT % K)],
                        acc.at[pl.ds(sid * ZPT + (ZPT // K) * K, ZPT % K)])
        pltpu.sync_copy(trs.at[pl.ds(0, 20)],
                        rs_acc.at[pl.ds(sid * 20, 20)])
        plsc.subcore_barrier()

        dcol = jnp.full((16,), head, jnp.int32)
        scol = jnp.full((16,), 2 + head, jnp.int32)

        def issue_idx(c, b):
            base = ebase + c * K
            pltpu.async_copy(ei_hbm.at[pl.ds(base, K)], src_b[b], sem_i[b])
            pltpu.async_copy(ei_hbm.at[pl.ds(E + base, K)], dst_b[b], sem_i[b])

        def wait_idx(c, b):
            base = ebase + c * K
            pltpu.make_async_copy(
                ei_hbm.at[pl.ds(base, K)], src_b[b], sem_i[b]).wait()
            pltpu.make_async_copy(
                ei_hbm.at[pl.ds(E + base, K)], dst_b[b], sem_i[b]).wait()

        def issue_gather(b):
            pltpu.async_copy(hext_hbm.at[dst_b[b]], rows_bf_b[b], sem_g[b])

        def compute(b):
            src_v, dst_v = src_b[b], dst_b[b]
            sidx_v, wbuf = sidx_b[b], wbuf_b[b]
            rows_bf, rows_v = rows_bf_b[b], rows_v_b[b]
            for g in range(K // 16):
                src16 = src_v[pl.ds(g * 16, 16)]
                dst16 = dst_v[pl.ds(g * 16, 16)]
                sd = plsc.load_gather(tab, [dst16, dcol])
                ss = plsc.load_gather(tab, [src16, scol])
                l = ss + sd
                w = jnp.exp(jnp.where(l >= 0.0, -l, -ALPHA * l))
                t = src16 - lo
                own = (t >= 0) & (t < NHALF)
                w = jnp.where(own, w, 0.0)
                sidx_v[pl.ds(g * 16, 16)] = jnp.where(own, t, NHALF)
                wbuf[pl.ds(g * 16, 16)] = w
                plsc.addupdate_scatter(
                    trs, [lax.shift_right_logical(src16, 4),
                          jnp.bitwise_and(src16, 15)], w)

            @plsc.parallel_loop(0, K, unroll=4)
            def _scale(e):
                ws = plsc.load_gather(wbuf, [jnp.full((16,), e, jnp.int32)])
                for c in range(CH // 32):
                    v = rows_bf[e, pl.ds(c * 32, 32)]
                    xi = plsc.bitcast(v, jnp.int32)
                    fe = plsc.bitcast(lax.shift_left(xi, 16), jnp.float32)
                    fo = plsc.bitcast(
                        jnp.bitwise_and(xi, jnp.int32(-65536)), jnp.float32)
                    rows_v[e, pl.ds(c * 32, 16)] = fe * ws
                    rows_v[e, pl.ds(c * 32 + 16, 16)] = fo * ws

        def wait_gather(b):
            pltpu.make_async_copy(
                hext_hbm.at[dst_b[b]], rows_bf_b[b], sem_g[b]).wait()

        def wait_scatter(b):
            pltpu.make_async_copy(
                rows_v_b[b], acc.at[sidx_b[b]], sem_s[b]).wait()

        def issue_scatter(b):
            pltpu.async_copy(rows_v_b[b], acc.at[sidx_b[b]], sem_s[b],
                             add=True)

        issue_idx(0, 0)
        wait_idx(0, 0)
        issue_gather(0)
        issue_idx(1, 1)

        def pipe(kk, carry):
            c0 = 2 * kk
            wait_gather(0)
            pl.when(kk > 0)(lambda: wait_scatter(0))
            compute(0)
            issue_scatter(0)
            wait_idx(c0 + 1, 1)
            issue_gather(1)
            pl.when(kk < NCHUNK // 2 - 1)(lambda: issue_idx(c0 + 2, 0))
            wait_gather(1)
            pl.when(kk > 0)(lambda: wait_scatter(1))
            compute(1)
            issue_scatter(1)

            def _next0():
                wait_idx(c0 + 2, 0)
                issue_gather(0)
                pl.when(kk < NCHUNK // 2 - 1)(lambda: issue_idx(c0 + 3, 1))
            pl.when(kk < NCHUNK // 2 - 1)(_next0)
            return carry
        lax.fori_loop(0, NCHUNK // 2, pipe, 0)
        wait_scatter(0)
        wait_scatter(1)
        plsc.subcore_barrier()

        for j in range(4):
            pltpu.sync_copy(trs.at[pl.ds(cid * (NHALF // 16) + j * 80, 80)],
                            rs_acc.at[ibuf.at[j]], add=True)
        plsc.subcore_barrier()

        pltpu.sync_copy(rs_acc.at[pl.ds(sid * 20, 20)],
                        trs.at[pl.ds(0, 20)])
        for z in range(WPT // K):
            r0 = sid * WPT + z * K
            pltpu.sync_copy(acc.at[pl.ds(r0, K)], rows_v_b[0])
            pltpu.sync_copy(rows_v_b[0], part_hbm.at[pl.ds(lo + r0, K)])
            for q in range(K // 16):
                rsv = trs[z * (K // 16) + q, pl.ds(0, 16)]
                plsc.store_scatter(
                    rsw_v, [jnp.full((16,), q * 16, jnp.int32) + lane,
                            jnp.zeros((16,), jnp.int32)], rsv)
            pltpu.sync_copy(rsw_v, rs_hbm.at[pl.ds(lo + r0, K)])

    return edge_kernel


def kernel(x, edge_index, n, W, b, a_param):
    nodes = x.shape[0]
    E = edge_index.shape[1]
    nh, ch, cin = W.shape

    wct = W.reshape(nh * ch, cin).T
    bc = b.reshape(1, nh * ch)
    a0, a1 = a_param[0], a_param[1]
    z = jnp.zeros((CH,), jnp.float32)
    cols = [
        jnp.concatenate([a0[CH:], z]),
        jnp.concatenate([z, a1[CH:]]),
        jnp.concatenate([a0[:CH], z]),
        jnp.concatenate([z, a1[:CH]]),
    ] + [jnp.zeros((CF,), jnp.float32)] * 28
    P = jnp.stack(cols, axis=1)

    perm = np.array([(m // 32) * 32 + (m % 32) // 2 + (16 if m % 2 else 0)
                     for m in range(CF)], dtype=np.int32)
    wct = wct[:, perm]
    bc = bc[:, perm]
    P = P[perm, :]
    x_pad = jnp.pad(x, ((0, N_PAD - nodes), (0, 0)))

    grid = N_PAD // ROWB
    h0, h1, tab = pl.pallas_call(
        _proj_body,
        grid=(grid,),
        in_specs=[
            pl.BlockSpec((ROWB, CF), lambda i: (i, 0)),
            pl.BlockSpec((CF, CF), lambda i: (0, 0)),
            pl.BlockSpec((1, CF), lambda i: (0, 0)),
            pl.BlockSpec((CF, 32), lambda i: (0, 0)),
        ],
        out_specs=[
            pl.BlockSpec((ROWB, CH), lambda i: (i, 0)),
            pl.BlockSpec((ROWB, CH), lambda i: (i, 0)),
            pl.BlockSpec((ROWB, 8), lambda i: (i, 0)),
        ],
        out_shape=[
            jax.ShapeDtypeStruct((N_PAD, CH), jnp.bfloat16),
            jax.ShapeDtypeStruct((N_PAD, CH), jnp.bfloat16),
            jax.ShapeDtypeStruct((N_PAD, 8), jnp.float32),
        ],
    )(x_pad, wct, bc, P)

    ei_flat = edge_index.reshape(-1)
    part0, rs0 = _make_edge_kernel(E, 0)(h0, tab, ei_flat)
    part1, rs1 = _make_edge_kernel(E, 1)(h1, tab, ei_flat)

    out_pad = pl.pallas_call(
        _final_body,
        grid=(grid,),
        in_specs=[
            pl.BlockSpec((ROWB, CH), lambda i: (i, 0)),
            pl.BlockSpec((ROWB, CH), lambda i: (i, 0)),
            pl.BlockSpec((ROWB, 16), lambda i: (i, 0)),
            pl.BlockSpec((ROWB, 16), lambda i: (i, 0)),
        ],
        out_specs=pl.BlockSpec((ROWB, CF), lambda i: (i, 0)),
        out_shape=jax.ShapeDtypeStruct((N_PAD, CF), jnp.float32),
    )(part0, part1, rs0, rs1)

    return out_pad[:nodes]
